# baseline (speedup 1.0000x reference)
import functools

import jax
import jax.numpy as jnp
from jax import lax
from jax.experimental import pallas as pl
from jax.experimental.pallas import tpu as pltpu
from jax.experimental.pallas import tpu_sc as plsc

_E = 8
_DIN = 1024
_DOUT = 1024
_BS = 256
_NW = 32
_CH = 64


def _make_row_gather(n_rows, n_cols, n_ch, name):
    per_w = n_rows // _NW
    chunk = per_w // n_ch
    assert per_w % n_ch == 0 and n_rows % _NW == 0
    assert chunk % 8 == 0 and chunk <= 128
    mesh = plsc.VectorSubcoreMesh(core_axis_name="c", subcore_axis_name="s")

    @functools.partial(
        pl.kernel,
        mesh=mesh,
        out_type=jax.ShapeDtypeStruct((n_rows, n_cols), jnp.float32),
        scratch_types=[
            pltpu.VMEM((per_w,), jnp.int32),
            pltpu.VMEM((chunk, n_cols), jnp.float32),
            pltpu.VMEM((chunk, n_cols), jnp.float32),
            pltpu.SemaphoreType.DMA,
            pltpu.SemaphoreType.DMA,
            pltpu.SemaphoreType.DMA,
            pltpu.SemaphoreType.DMA,
        ],
        name=name,
    )
    def gather(table_hbm, idx_hbm, out_hbm, idx_v, buf0, buf1,
               gs0, gs1, os0, os1):
        wid = lax.axis_index("s") * 2 + lax.axis_index("c")
        base = wid * per_w
        bufs, gsems, osems = (buf0, buf1), (gs0, gs1), (os0, os1)

        pltpu.sync_copy(idx_hbm.at[pl.ds(base, per_w)], idx_v)

        def start_gather(c, b):
            return pltpu.async_copy(
                table_hbm.at[idx_v.at[pl.ds(c * chunk, chunk)]],
                bufs[b], gsems[b])

        def start_out(c, b):
            return pltpu.async_copy(
                bufs[b], out_hbm.at[pl.ds(base + c * chunk, chunk)], osems[b])

        gh = [start_gather(0, 0), None]
        oh = [None, None]
        for c in range(n_ch):
            b = c & 1
            nb = 1 - b
            if c + 1 < n_ch:
                if oh[nb] is not None:
                    oh[nb].wait()
                gh[nb] = start_gather(c + 1, nb)
            gh[b].wait()
            oh[b] = start_out(c, b)
        for h in oh:
            if h is not None:
                h.wait()

    return gather


def _make_row_scatter(n_rows_in, n_rows_out, n_cols, n_ch, name,
                      dtype=jnp.float32):
    per_w = n_rows_in // _NW
    chunk = per_w // n_ch
    assert per_w % n_ch == 0 and n_rows_in % _NW == 0
    assert chunk % 8 == 0 and chunk <= 128
    mesh = plsc.VectorSubcoreMesh(core_axis_name="c", subcore_axis_name="s")

    @functools.partial(
        pl.kernel,
        mesh=mesh,
        out_type=jax.ShapeDtypeStruct((n_rows_out, n_cols), dtype),
        scratch_types=[
            pltpu.VMEM((n_ch, chunk), jnp.int32),
            pltpu.VMEM((chunk, n_cols), dtype),
            pltpu.VMEM((chunk, n_cols), dtype),
            pltpu.SemaphoreType.DMA,
            pltpu.SemaphoreType.DMA,
            pltpu.SemaphoreType.DMA,
            pltpu.SemaphoreType.DMA,
        ],
        name=name,
    )
    def scatter(x_hbm, idx3_hbm, out_hbm, idx_v, buf0, buf1,
                rs0, rs1, ws0, ws1):
        wid = lax.axis_index("s") * 2 + lax.axis_index("c")
        base = wid * per_w
        bufs, rsems, wsems = (buf0, buf1), (rs0, rs1), (ws0, ws1)

        pltpu.sync_copy(idx3_hbm.at[wid], idx_v)

        def start_read(c, b):
            return pltpu.async_copy(
                x_hbm.at[pl.ds(base + c * chunk, chunk)], bufs[b], rsems[b])

        def start_write(c, b):
            return pltpu.async_copy(
                bufs[b], out_hbm.at[idx_v.at[c]], wsems[b])

        rh = [start_read(0, 0), None]
        wh = [None, None]
        for c in range(n_ch):
            b = c & 1
            nb = 1 - b
            if c + 1 < n_ch:
                if wh[nb] is not None:
                    wh[nb].wait()
                rh[nb] = start_read(c + 1, nb)
            rh[b].wait()
            wh[b] = start_write(c, b)
        for h in wh:
            if h is not None:
                h.wait()

    return scatter


def _mm_body(be_ref, x_ref, w_ref, b_ref, o_ref):
    w = w_ref[...]
    y = lax.dot_general(
        x_ref[...], w, (((1,), (1,)), ((), ())),
        preferred_element_type=jnp.float32)
    o_ref[...] = y + b_ref[0]


def _grouped_matmul(x_sorted, w2d, bias_t, block_expert):
    n_pad = x_sorted.shape[0]
    nb = n_pad // _BS
    grid_spec = pltpu.PrefetchScalarGridSpec(
        num_scalar_prefetch=1,
        grid=(nb,),
        in_specs=[
            pl.BlockSpec((_BS, _DIN), lambda i, be: (i, 0)),
            pl.BlockSpec((_DOUT, _DIN), lambda i, be: (0, be[i])),
            pl.BlockSpec((1, 1, _DOUT), lambda i, be: (be[i], 0, 0)),
        ],
        out_specs=pl.BlockSpec((_BS, _DOUT), lambda i, be: (i, 0)),
    )
    return pl.pallas_call(
        _mm_body,
        grid_spec=grid_spec,
        out_shape=jax.ShapeDtypeStruct((n_pad, _DOUT), jnp.float32),
        name="grouped_mm",
    )(block_expert, x_sorted, w2d, bias_t)


def kernel(input, topk_indices, expert_weights, expert_biases):
    b, s, k, din = input.shape
    n = b * s * k
    n_pad = n + _E * _BS
    nb = n_pad // _BS

    x = input.reshape(n, din)
    idx = topk_indices.reshape(n).astype(jnp.int32)

    onehot = (idx[:, None] == jnp.arange(_E, dtype=jnp.int32)[None, :])
    csum = jnp.cumsum(onehot.astype(jnp.int32), axis=0)
    counts = csum[-1]
    rank = jnp.take_along_axis(csum, idx[:, None], axis=1)[:, 0] - 1
    padded = ((counts + _BS - 1) // _BS) * _BS
    ends = jnp.cumsum(padded)
    starts = ends - padded
    dest = starts[idx] + rank
    block_expert = jnp.minimum(
        jnp.searchsorted(ends, jnp.arange(nb, dtype=jnp.int32) * _BS,
                         side="right"),
        _E - 1).astype(jnp.int32)

    n_ch = 8
    dest3 = dest.reshape(_NW, n_ch, n // (_NW * n_ch))
    x_sorted = _make_row_scatter(n, n_pad, din, n_ch,
                                 "scatter_tokens")(x, dest3)
    x_sorted = x_sorted.astype(jnp.bfloat16)

    w2d = expert_weights.astype(jnp.bfloat16).reshape(_DOUT, _E * _DIN)
    bias_t = expert_biases[:, :, 0].T.reshape(_E, 1, _DOUT)
    y_sorted = _grouped_matmul(x_sorted, w2d, bias_t, block_expert)

    out = _make_row_gather(n, _DOUT, 8, "ungather_out")(y_sorted, dest)
    return out.reshape(b, s, k, _DOUT)


# speedup vs baseline: 1.0408x; 1.0408x over previous
import functools

import jax
import jax.numpy as jnp
from jax import lax
from jax.experimental import pallas as pl
from jax.experimental.pallas import tpu as pltpu
from jax.experimental.pallas import tpu_sc as plsc

_E = 8
_DIN = 1024
_DOUT = 1024
_BS = 256
_NW = 32
_CH = 64


def _make_row_gather(n_rows, n_cols, n_ch, name):
    per_w = n_rows // _NW
    chunk = per_w // n_ch
    assert per_w % n_ch == 0 and n_rows % _NW == 0
    assert chunk % 8 == 0 and chunk <= 128
    mesh = plsc.VectorSubcoreMesh(core_axis_name="c", subcore_axis_name="s")

    @functools.partial(
        pl.kernel,
        mesh=mesh,
        out_type=jax.ShapeDtypeStruct((n_rows, n_cols), jnp.float32),
        scratch_types=[
            pltpu.VMEM((per_w,), jnp.int32),
            pltpu.VMEM((chunk, n_cols), jnp.float32),
            pltpu.VMEM((chunk, n_cols), jnp.float32),
            pltpu.SemaphoreType.DMA,
            pltpu.SemaphoreType.DMA,
            pltpu.SemaphoreType.DMA,
            pltpu.SemaphoreType.DMA,
        ],
        name=name,
    )
    def gather(table_hbm, idx_hbm, out_hbm, idx_v, buf0, buf1,
               gs0, gs1, os0, os1):
        wid = lax.axis_index("s") * 2 + lax.axis_index("c")
        base = wid * per_w
        bufs, gsems, osems = (buf0, buf1), (gs0, gs1), (os0, os1)

        pltpu.sync_copy(idx_hbm.at[pl.ds(base, per_w)], idx_v)

        def start_gather(c, b):
            return pltpu.async_copy(
                table_hbm.at[idx_v.at[pl.ds(c * chunk, chunk)]],
                bufs[b], gsems[b])

        def start_out(c, b):
            return pltpu.async_copy(
                bufs[b], out_hbm.at[pl.ds(base + c * chunk, chunk)], osems[b])

        gh = [start_gather(0, 0), None]
        oh = [None, None]
        for c in range(n_ch):
            b = c & 1
            nb = 1 - b
            if c + 1 < n_ch:
                if oh[nb] is not None:
                    oh[nb].wait()
                gh[nb] = start_gather(c + 1, nb)
            gh[b].wait()
            oh[b] = start_out(c, b)
        for h in oh:
            if h is not None:
                h.wait()

    return gather


def _make_row_scatter(n_rows_in, n_rows_out, n_cols, n_ch, name,
                      dtype=jnp.float32):
    per_w = n_rows_in // _NW
    chunk = per_w // n_ch
    assert per_w % n_ch == 0 and n_rows_in % _NW == 0
    assert chunk % 8 == 0 and chunk <= 128
    mesh = plsc.VectorSubcoreMesh(core_axis_name="c", subcore_axis_name="s")

    @functools.partial(
        pl.kernel,
        mesh=mesh,
        out_type=jax.ShapeDtypeStruct((n_rows_out, n_cols), dtype),
        scratch_types=[
            pltpu.VMEM((n_ch, chunk), jnp.int32),
            pltpu.VMEM((chunk, n_cols), dtype),
            pltpu.VMEM((chunk, n_cols), dtype),
            pltpu.SemaphoreType.DMA,
            pltpu.SemaphoreType.DMA,
            pltpu.SemaphoreType.DMA,
            pltpu.SemaphoreType.DMA,
        ],
        name=name,
    )
    def scatter(x_hbm, idx3_hbm, out_hbm, idx_v, buf0, buf1,
                rs0, rs1, ws0, ws1):
        wid = lax.axis_index("s") * 2 + lax.axis_index("c")
        base = wid * per_w
        bufs, rsems, wsems = (buf0, buf1), (rs0, rs1), (ws0, ws1)

        pltpu.sync_copy(idx3_hbm.at[wid], idx_v)

        def start_read(c, b):
            return pltpu.async_copy(
                x_hbm.at[pl.ds(base + c * chunk, chunk)], bufs[b], rsems[b])

        def start_write(c, b):
            return pltpu.async_copy(
                bufs[b], out_hbm.at[idx_v.at[c]], wsems[b])

        rh = [start_read(0, 0), None]
        wh = [None, None]
        for c in range(n_ch):
            b = c & 1
            nb = 1 - b
            if c + 1 < n_ch:
                if wh[nb] is not None:
                    wh[nb].wait()
                rh[nb] = start_read(c + 1, nb)
            rh[b].wait()
            wh[b] = start_write(c, b)
        for h in wh:
            if h is not None:
                h.wait()

    return scatter


def _mm_body(be_ref, x_ref, w_ref, b_ref, o_ref):
    e = be_ref[pl.program_id(0)]
    w = w_ref[:, e, :]
    y = lax.dot_general(
        x_ref[...], w, (((1,), (1,)), ((), ())),
        preferred_element_type=jnp.float32)
    o_ref[...] = y + b_ref[0]


def _grouped_matmul(x_sorted, expert_weights, bias_t, block_expert):
    n_pad = x_sorted.shape[0]
    nb = n_pad // _BS
    grid_spec = pltpu.PrefetchScalarGridSpec(
        num_scalar_prefetch=1,
        grid=(nb,),
        in_specs=[
            pl.BlockSpec((_BS, _DIN), lambda i, be: (i, 0)),
            pl.BlockSpec((_DOUT, _E, _DIN), lambda i, be: (0, 0, 0)),
            pl.BlockSpec((1, 1, _DOUT), lambda i, be: (be[i], 0, 0)),
        ],
        out_specs=pl.BlockSpec((_BS, _DOUT), lambda i, be: (i, 0)),
    )
    return pl.pallas_call(
        _mm_body,
        grid_spec=grid_spec,
        out_shape=jax.ShapeDtypeStruct((n_pad, _DOUT), jnp.float32),
        name="grouped_mm",
    )(block_expert, x_sorted, expert_weights, bias_t)


def kernel(input, topk_indices, expert_weights, expert_biases):
    b, s, k, din = input.shape
    n = b * s * k
    n_pad = n + _E * _BS
    nb = n_pad // _BS

    x = input.reshape(n, din)
    idx = topk_indices.reshape(n).astype(jnp.int32)

    onehot = (idx[:, None] == jnp.arange(_E, dtype=jnp.int32)[None, :])
    csum = jnp.cumsum(onehot.astype(jnp.int32), axis=0)
    counts = csum[-1]
    rank = jnp.take_along_axis(csum, idx[:, None], axis=1)[:, 0] - 1
    padded = ((counts + _BS - 1) // _BS) * _BS
    ends = jnp.cumsum(padded)
    starts = ends - padded
    dest = starts[idx] + rank
    block_starts = jnp.arange(nb, dtype=jnp.int32) * _BS
    block_expert = jnp.minimum(
        jnp.sum((block_starts[:, None] >= ends[None, :]).astype(jnp.int32),
                axis=1),
        _E - 1).astype(jnp.int32)

    n_ch = 8
    dest3 = dest.reshape(_NW, n_ch, n // (_NW * n_ch))
    x_sorted = _make_row_scatter(n, n_pad, din, n_ch,
                                 "scatter_tokens")(x, dest3)

    bias_t = expert_biases[:, :, 0].T.reshape(_E, 1, _DOUT)
    y_sorted = _grouped_matmul(x_sorted, expert_weights, bias_t, block_expert)

    out = _make_row_gather(n, _DOUT, 8, "ungather_out")(y_sorted, dest)
    return out.reshape(b, s, k, _DOUT)


# speedup vs baseline: 1.1276x; 1.0834x over previous
import functools

import jax
import jax.numpy as jnp
from jax import lax
from jax.experimental import pallas as pl
from jax.experimental.pallas import tpu as pltpu
from jax.experimental.pallas import tpu_sc as plsc

_E = 8
_DIN = 1024
_DOUT = 1024
_BS = 256
_NW = 32
_CH = 64


def _make_row_gather(n_rows, n_cols, n_ch, name):
    per_w = n_rows // _NW
    chunk = per_w // n_ch
    assert per_w % n_ch == 0 and n_rows % _NW == 0
    assert chunk % 8 == 0 and chunk <= 128
    mesh = plsc.VectorSubcoreMesh(core_axis_name="c", subcore_axis_name="s")

    @functools.partial(
        pl.kernel,
        mesh=mesh,
        out_type=jax.ShapeDtypeStruct((n_rows, n_cols), jnp.float32),
        scratch_types=[
            pltpu.VMEM((per_w,), jnp.int32),
            pltpu.VMEM((chunk, n_cols), jnp.float32),
            pltpu.VMEM((chunk, n_cols), jnp.float32),
            pltpu.SemaphoreType.DMA,
            pltpu.SemaphoreType.DMA,
            pltpu.SemaphoreType.DMA,
            pltpu.SemaphoreType.DMA,
        ],
        name=name,
    )
    def gather(table_hbm, idx_hbm, out_hbm, idx_v, buf0, buf1,
               gs0, gs1, os0, os1):
        wid = lax.axis_index("s") * 2 + lax.axis_index("c")
        base = wid * per_w
        bufs, gsems, osems = (buf0, buf1), (gs0, gs1), (os0, os1)

        pltpu.sync_copy(idx_hbm.at[pl.ds(base, per_w)], idx_v)

        def start_gather(c, b):
            return pltpu.async_copy(
                table_hbm.at[idx_v.at[pl.ds(c * chunk, chunk)]],
                bufs[b], gsems[b])

        def start_out(c, b):
            return pltpu.async_copy(
                bufs[b], out_hbm.at[pl.ds(base + c * chunk, chunk)], osems[b])

        gh = [start_gather(0, 0), None]
        oh = [None, None]
        for c in range(n_ch):
            b = c & 1
            nb = 1 - b
            if c + 1 < n_ch:
                if oh[nb] is not None:
                    oh[nb].wait()
                gh[nb] = start_gather(c + 1, nb)
            gh[b].wait()
            oh[b] = start_out(c, b)
        for h in oh:
            if h is not None:
                h.wait()

    return gather


def _make_row_scatter(n_rows_in, n_rows_out, n_cols, n_ch, name,
                      dtype=jnp.float32):
    per_w = n_rows_in // _NW
    chunk = per_w // n_ch
    assert per_w % n_ch == 0 and n_rows_in % _NW == 0
    assert chunk % 8 == 0 and chunk <= 128
    mesh = plsc.VectorSubcoreMesh(core_axis_name="c", subcore_axis_name="s")

    @functools.partial(
        pl.kernel,
        mesh=mesh,
        out_type=jax.ShapeDtypeStruct((n_rows_out, n_cols), dtype),
        scratch_types=[
            pltpu.VMEM((n_ch, chunk), jnp.int32),
            pltpu.VMEM((chunk, n_cols), dtype),
            pltpu.VMEM((chunk, n_cols), dtype),
            pltpu.SemaphoreType.DMA,
            pltpu.SemaphoreType.DMA,
            pltpu.SemaphoreType.DMA,
            pltpu.SemaphoreType.DMA,
        ],
        name=name,
    )
    def scatter(x_hbm, idx3_hbm, out_hbm, idx_v, buf0, buf1,
                rs0, rs1, ws0, ws1):
        wid = lax.axis_index("s") * 2 + lax.axis_index("c")
        base = wid * per_w
        bufs, rsems, wsems = (buf0, buf1), (rs0, rs1), (ws0, ws1)

        pltpu.sync_copy(idx3_hbm.at[wid], idx_v)

        def start_read(c, b):
            return pltpu.async_copy(
                x_hbm.at[pl.ds(base + c * chunk, chunk)], bufs[b], rsems[b])

        def start_write(c, b):
            return pltpu.async_copy(
                bufs[b], out_hbm.at[idx_v.at[c]], wsems[b])

        rh = [start_read(0, 0), None]
        wh = [None, None]
        for c in range(n_ch):
            b = c & 1
            nb = 1 - b
            if c + 1 < n_ch:
                if wh[nb] is not None:
                    wh[nb].wait()
                rh[nb] = start_read(c + 1, nb)
            rh[b].wait()
            wh[b] = start_write(c, b)
        for h in wh:
            if h is not None:
                h.wait()

    return scatter


def _mm_body(be_ref, x_ref, w_ref, b_ref, o_ref):
    w = w_ref[...]
    y = lax.dot_general(
        x_ref[...], w, (((1,), (1,)), ((), ())),
        preferred_element_type=jnp.float32)
    o_ref[...] = y + b_ref[0]


def _grouped_matmul(x_sorted, w2d, bias_t, block_expert):
    n_pad = x_sorted.shape[0]
    nb = n_pad // _BS
    grid_spec = pltpu.PrefetchScalarGridSpec(
        num_scalar_prefetch=1,
        grid=(nb,),
        in_specs=[
            pl.BlockSpec((_BS, _DIN), lambda i, be: (i, 0)),
            pl.BlockSpec((_DOUT, _DIN), lambda i, be: (0, be[i])),
            pl.BlockSpec((1, 1, _DOUT), lambda i, be: (be[i], 0, 0)),
        ],
        out_specs=pl.BlockSpec((_BS, _DOUT), lambda i, be: (i, 0)),
    )
    return pl.pallas_call(
        _mm_body,
        grid_spec=grid_spec,
        out_shape=jax.ShapeDtypeStruct((n_pad, _DOUT), jnp.float32),
        name="grouped_mm",
    )(block_expert, x_sorted, w2d, bias_t)


def kernel(input, topk_indices, expert_weights, expert_biases):
    b, s, k, din = input.shape
    n = b * s * k
    n_pad = n + _E * _BS
    nb = n_pad // _BS

    x = input.reshape(n, din)
    idx = topk_indices.reshape(n).astype(jnp.int32)

    onehot = (idx[:, None] == jnp.arange(_E, dtype=jnp.int32)[None, :])
    csum = jnp.cumsum(onehot.astype(jnp.int32), axis=0)
    counts = csum[-1]
    rank = jnp.take_along_axis(csum, idx[:, None], axis=1)[:, 0] - 1
    padded = ((counts + _BS - 1) // _BS) * _BS
    ends = jnp.cumsum(padded)
    starts = ends - padded
    dest = starts[idx] + rank
    block_starts = jnp.arange(nb, dtype=jnp.int32) * _BS
    block_expert = jnp.minimum(
        jnp.sum((block_starts[:, None] >= ends[None, :]).astype(jnp.int32),
                axis=1),
        _E - 1).astype(jnp.int32)

    n_ch = 8
    dest3 = dest.reshape(_NW, n_ch, n // (_NW * n_ch))
    x_sorted = _make_row_scatter(n, n_pad, din, n_ch,
                                 "scatter_tokens")(x, dest3)

    w2d = expert_weights.reshape(_DOUT, _E * _DIN)
    bias_t = expert_biases[:, :, 0].T.reshape(_E, 1, _DOUT)
    y_sorted = _grouped_matmul(x_sorted, w2d, bias_t, block_expert)

    out = _make_row_gather(n, _DOUT, 8, "ungather_out")(y_sorted, dest)
    return out.reshape(b, s, k, _DOUT)
